# Initial kernel scaffold; baseline (speedup 1.0000x reference)
import jax
import jax.numpy as jnp
from jax.experimental import pallas as pl


def kernel(x, pos, W_enc, b_enc, Wqkv, Wp1, bp1, Wp2, bp2, Wa1, ba1, Wa2, ba2, W_fc, b_fc):
    raise NotImplementedError("write your pallas kernel here")


# speedup vs baseline: 17.0628x; 17.0628x over previous
import functools

import jax
import jax.numpy as jnp
from jax import lax
from jax.experimental import pallas as pl
from jax.experimental.pallas import tpu as pltpu
from jax.experimental.pallas import tpu_sc as plsc

B, N, K = 2, 2048, 3
DIM, IN_F = 128, 128
POS_H, ATT_H = 64, 512
L = 3
BN = B * N
E = K * BN
PPAD = 128

F32 = jnp.float32

QB = 256
NQ = N // QB


def _knn_body(posq_ref, posc_ref, idx_ref):
    b = pl.program_id(0)
    q = posq_ref[...]
    pc = posc_ref[0]
    d = None
    for c in range(3):
        rel = q[:, c:c + 1] - pc[c:c + 1, :]
        sq = rel * rel
        d = sq if d is None else d + sq
    iota = lax.broadcasted_iota(jnp.int32, (QB, N), 1)
    lane = lax.broadcasted_iota(jnp.int32, (QB, 128), 1)
    base = b * N
    acc = jnp.zeros((QB, 128), jnp.int32)
    for t in range(K):
        m = jnp.min(d, axis=1, keepdims=True)
        cand = jnp.where(d <= m, iota, N)
        it = jnp.min(cand, axis=1, keepdims=True)
        d = jnp.where(iota == it, jnp.float32(jnp.inf), d)
        acc = acc + jnp.where(lane == t, it + base, 0)
    idx_ref[...] = acc


_knn_call = pl.pallas_call(
    _knn_body,
    grid=(B, NQ),
    in_specs=[
        pl.BlockSpec((QB, PPAD), lambda b, i: (b * NQ + i, 0)),
        pl.BlockSpec((1, 8, N), lambda b, i: (b, 0, 0)),
    ],
    out_specs=pl.BlockSpec((QB, 128), lambda b, i: (b * NQ + i, 0)),
    out_shape=jax.ShapeDtypeStruct((BN, 128), jnp.int32),
)

QE = 512
NE = BN // QE


def _enc_body(x_ref, we_ref, be_ref, wq_ref, wk_ref, wv_ref,
              q_ref, k_ref, v_ref):
    h = jnp.dot(x_ref[...], we_ref[...], preferred_element_type=F32) + be_ref[...]
    q_ref[...] = jnp.dot(h, wq_ref[...], preferred_element_type=F32)
    k_ref[...] = jnp.dot(h, wk_ref[...], preferred_element_type=F32)
    v_ref[...] = jnp.dot(h, wv_ref[...], preferred_element_type=F32)


_enc_call = pl.pallas_call(
    _enc_body,
    grid=(NE,),
    in_specs=[
        pl.BlockSpec((QE, IN_F), lambda i: (i, 0)),
        pl.BlockSpec((IN_F, DIM), lambda i: (0, 0)),
        pl.BlockSpec((1, DIM), lambda i: (0, 0)),
        pl.BlockSpec((DIM, DIM), lambda i: (0, 0)),
        pl.BlockSpec((DIM, DIM), lambda i: (0, 0)),
        pl.BlockSpec((DIM, DIM), lambda i: (0, 0)),
    ],
    out_specs=[
        pl.BlockSpec((QE, DIM), lambda i: (i, 0)),
        pl.BlockSpec((QE, DIM), lambda i: (i, 0)),
        pl.BlockSpec((QE, DIM), lambda i: (i, 0)),
    ],
    out_shape=[jax.ShapeDtypeStruct((BN, DIM), F32)] * 3,
)

NC, NS = 2, 16
NW = NC * NS
RPW = E // NW
CH = RPW // 128

@functools.lru_cache(maxsize=None)
def _make_gather(with_pos):
    _sc_mesh = plsc.VectorSubcoreMesh(core_axis_name="c", subcore_axis_name="s",
                                      num_cores=NC, num_subcores=NS)

    def body(*refs):
        if with_pos:
            ptab, idx_hbm, pg, idx_v, pbuf, sem = refs
            tabs, bufs, outs = [ptab], [pbuf], [pg]
        else:
            ktab, vtab, idx_hbm, kg, vg, idx_v, kbuf, vbuf, sem = refs
            tabs, bufs, outs = [ktab, vtab], [kbuf, vbuf], [kg, vg]
        wid = lax.axis_index("s") * NC + lax.axis_index("c")
        row0 = wid * RPW
        pltpu.sync_copy(idx_hbm.at[wid], idx_v)
        copies = []
        for j in range(CH):
            ij = idx_v.at[j]
            dst = pl.ds(j * 128, 128)
            for tab, buf in zip(tabs, bufs):
                copies.append(pltpu.async_copy(tab.at[ij], buf.at[dst], sem))
        for cp in copies:
            cp.wait()
        for buf, out in zip(bufs, outs):
            pltpu.sync_copy(buf, out.at[pl.ds(row0, RPW)])

    n = 1 if with_pos else 2
    out_type = [jax.ShapeDtypeStruct((E, DIM), F32)] * n
    scratch = ([pltpu.VMEM((8, 128), jnp.int32)]
               + [pltpu.VMEM((RPW, DIM), F32)] * n
               + [pltpu.SemaphoreType.DMA])
    return pl.kernel(body, out_type=out_type, mesh=_sc_mesh,
                     scratch_types=scratch)


def _gather_pos(p, idx2d):
    return _make_gather(True)(p, idx2d)[0]


def _gather_kv(k, v, idx2d):
    return _make_gather(False)(k, v, idx2d)

QL = 512
NL = BN // QL


def _layer_body(emit_qkv, *refs):
    if emit_qkv:
        (q_ref, kg_ref, vg_ref, pg_ref, posr_ref,
         wp1_ref, bp1_ref, wp2_ref, bp2_ref,
         wa1_ref, ba1_ref, wa2_ref, ba2_ref,
         wq_ref, wk_ref, wv_ref,
         qo_ref, ko_ref, vo_ref) = refs
    else:
        (q_ref, kg_ref, vg_ref, pg_ref, posr_ref,
         wp1_ref, bp1_ref, wp2_ref, bp2_ref,
         wa1_ref, ba1_ref, wa2_ref, ba2_ref,
         h_ref) = refs
    q = q_ref[...]
    posr = posr_ref[...]
    wp1 = wp1_ref[...]
    bp1 = bp1_ref[...]
    wp2 = wp2_ref[...]
    bp2 = bp2_ref[...]
    wa1 = wa1_ref[...]
    ba1 = ba1_ref[...]
    wa2 = wa2_ref[...]
    ba2 = ba2_ref[...]
    sims = []
    ves = []
    for t in range(K):
        rel = posr - pg_ref[t]
        p1 = jnp.maximum(
            jnp.dot(rel, wp1, preferred_element_type=F32) + bp1, 0.0)
        pe = jnp.dot(p1, wp2, preferred_element_type=F32) + bp2
        u = q - kg_ref[t] + pe
        s = jnp.maximum(
            jnp.dot(u, wa1, preferred_element_type=F32) + ba1, 0.0)
        sims.append(jnp.dot(s, wa2, preferred_element_type=F32) + ba2)
        ves.append(vg_ref[t] + pe)
    m = jnp.maximum(jnp.maximum(sims[0], sims[1]), sims[2])
    es = [jnp.exp(sv - m) for sv in sims]
    den = es[0] + es[1] + es[2]
    h = (es[0] * ves[0] + es[1] * ves[1] + es[2] * ves[2]) / den
    if emit_qkv:
        qo_ref[...] = jnp.dot(h, wq_ref[...], preferred_element_type=F32)
        ko_ref[...] = jnp.dot(h, wk_ref[...], preferred_element_type=F32)
        vo_ref[...] = jnp.dot(h, wv_ref[...], preferred_element_type=F32)
    else:
        h_ref[...] = h


def _make_layer(emit_qkv):
    in_specs = [
        pl.BlockSpec((QL, DIM), lambda i: (i, 0)),
        pl.BlockSpec((K, QL, DIM), lambda i: (0, i, 0)),
        pl.BlockSpec((K, QL, DIM), lambda i: (0, i, 0)),
        pl.BlockSpec((K, QL, PPAD), lambda i: (0, i, 0)),
        pl.BlockSpec((QL, PPAD), lambda i: (i, 0)),
        pl.BlockSpec((PPAD, POS_H), lambda i: (0, 0)),
        pl.BlockSpec((1, POS_H), lambda i: (0, 0)),
        pl.BlockSpec((POS_H, DIM), lambda i: (0, 0)),
        pl.BlockSpec((1, DIM), lambda i: (0, 0)),
        pl.BlockSpec((DIM, ATT_H), lambda i: (0, 0)),
        pl.BlockSpec((1, ATT_H), lambda i: (0, 0)),
        pl.BlockSpec((ATT_H, DIM), lambda i: (0, 0)),
        pl.BlockSpec((1, DIM), lambda i: (0, 0)),
    ]
    if emit_qkv:
        in_specs += [pl.BlockSpec((DIM, DIM), lambda i: (0, 0))] * 3
        out_specs = [pl.BlockSpec((QL, DIM), lambda i: (i, 0))] * 3
        out_shape = [jax.ShapeDtypeStruct((BN, DIM), F32)] * 3
    else:
        out_specs = pl.BlockSpec((QL, DIM), lambda i: (i, 0))
        out_shape = jax.ShapeDtypeStruct((BN, DIM), F32)
    return pl.pallas_call(
        functools.partial(_layer_body, emit_qkv),
        grid=(NL,),
        in_specs=in_specs,
        out_specs=out_specs,
        out_shape=out_shape,
    )


_layer_mid = _make_layer(True)
_layer_last = _make_layer(False)


def _pool_body(h_ref, wfc_ref, bfc_ref, out_ref):
    hs = jnp.sum(h_ref[0], axis=0, keepdims=True) * (1.0 / N)
    out_ref[0] = (jnp.dot(hs, wfc_ref[...], preferred_element_type=F32)
                  + bfc_ref[...])


_pool_call = pl.pallas_call(
    _pool_body,
    grid=(B,),
    in_specs=[
        pl.BlockSpec((1, N, DIM), lambda b: (b, 0, 0)),
        pl.BlockSpec((DIM, 128), lambda b: (0, 0)),
        pl.BlockSpec((1, 128), lambda b: (0, 0)),
    ],
    out_specs=pl.BlockSpec((1, 1, 128), lambda b: (b, 0, 0)),
    out_shape=jax.ShapeDtypeStruct((B, 1, 128), F32),
)


def kernel(x, pos, W_enc, b_enc, Wqkv, Wp1, bp1, Wp2, bp2, Wa1, ba1,
           Wa2, ba2, W_fc, b_fc):
    posr = jnp.pad(pos.reshape(BN, 3), ((0, 0), (0, PPAD - 3)))
    posc = jnp.pad(jnp.transpose(pos, (0, 2, 1)),
                   ((0, 0), (0, 5), (0, 0)))
    idx_cols = _knn_call(posr, posc)
    idx2d = jnp.pad(idx_cols[:, :K].T.reshape(NW, CH, 128),
                    ((0, 0), (0, 8 - CH), (0, 0)))

    Wq = Wqkv[:, :, :DIM]
    Wk = Wqkv[:, :, DIM:2 * DIM]
    Wv = Wqkv[:, :, 2 * DIM:]
    Wp1p = jnp.pad(Wp1, ((0, 0), (0, PPAD - 3), (0, 0)))

    q, k, v = _enc_call(x.reshape(BN, IN_F), W_enc, b_enc.reshape(1, DIM),
                        Wq[0], Wk[0], Wv[0])

    pg = _gather_pos(posr, idx2d)
    for l in range(L):
        kg, vg = _gather_kv(k, v, idx2d)
        args = (q, kg.reshape(K, BN, DIM), vg.reshape(K, BN, DIM),
                pg.reshape(K, BN, PPAD), posr,
                Wp1p[l], bp1[l].reshape(1, POS_H), Wp2[l],
                bp2[l].reshape(1, DIM), Wa1[l], ba1[l].reshape(1, ATT_H),
                Wa2[l], ba2[l].reshape(1, DIM))
        if l < L - 1:
            q, k, v = _layer_mid(*(args + (Wq[l + 1], Wk[l + 1], Wv[l + 1])))
        else:
            h = _layer_last(*args)

    wfc_pad = jnp.pad(W_fc, ((0, 0), (0, 128 - 1)))
    bfc_pad = jnp.pad(b_fc.reshape(1, 1), ((0, 0), (0, 128 - 1)))
    pooled = _pool_call(h.reshape(B, N, DIM), wfc_pad, bfc_pad)
    return pooled[:, 0, 0:1]


# speedup vs baseline: 18.5921x; 1.0896x over previous
import functools

import jax
import jax.numpy as jnp
from jax import lax
from jax.experimental import pallas as pl
from jax.experimental.pallas import tpu as pltpu
from jax.experimental.pallas import tpu_sc as plsc

B, N, K = 2, 2048, 3
DIM, IN_F = 128, 128
POS_H, ATT_H = 64, 512
L = 3
BN = B * N
E = K * BN
PPAD = 128

F32 = jnp.float32

QB = 256
NQ = N // QB


def _knn_body(posq_ref, posc_ref, x_ref, we_ref, be_ref, wq_ref, wk_ref,
              wv_ref, idx_ref, q_ref, k_ref, v_ref):
    b = pl.program_id(0)
    q = posq_ref[...]
    pc = posc_ref[0]
    d = None
    for c in range(3):
        rel = q[:, c:c + 1] - pc[c:c + 1, :]
        sq = rel * rel
        d = sq if d is None else d + sq
    iota = lax.broadcasted_iota(jnp.int32, (QB, N), 1)
    lane = lax.broadcasted_iota(jnp.int32, (QB, 128), 1)
    base = b * N
    acc = jnp.zeros((QB, 128), jnp.int32)
    for t in range(K):
        it = jnp.argmin(d, axis=1).reshape(QB, 1)
        d = jnp.where(iota == it, jnp.float32(jnp.inf), d)
        acc = acc + jnp.where(lane == t, it + base, 0)
    idx_ref[...] = acc
    h = (jnp.dot(x_ref[...], we_ref[...], preferred_element_type=F32)
         + be_ref[...])
    q_ref[...] = jnp.dot(h, wq_ref[...], preferred_element_type=F32)
    k_ref[...] = jnp.dot(h, wk_ref[...], preferred_element_type=F32)
    v_ref[...] = jnp.dot(h, wv_ref[...], preferred_element_type=F32)


_knn_call = pl.pallas_call(
    _knn_body,
    grid=(B, NQ),
    in_specs=[
        pl.BlockSpec((QB, 3), lambda b, i: (b * NQ + i, 0)),
        pl.BlockSpec((1, 3, N), lambda b, i: (b, 0, 0)),
        pl.BlockSpec((QB, IN_F), lambda b, i: (b * NQ + i, 0)),
        pl.BlockSpec((IN_F, DIM), lambda b, i: (0, 0)),
        pl.BlockSpec((1, DIM), lambda b, i: (0, 0)),
        pl.BlockSpec((DIM, DIM), lambda b, i: (0, 0)),
        pl.BlockSpec((DIM, DIM), lambda b, i: (0, 0)),
        pl.BlockSpec((DIM, DIM), lambda b, i: (0, 0)),
    ],
    out_specs=[
        pl.BlockSpec((QB, 128), lambda b, i: (b * NQ + i, 0)),
        pl.BlockSpec((QB, DIM), lambda b, i: (b * NQ + i, 0)),
        pl.BlockSpec((QB, DIM), lambda b, i: (b * NQ + i, 0)),
        pl.BlockSpec((QB, DIM), lambda b, i: (b * NQ + i, 0)),
    ],
    out_shape=[jax.ShapeDtypeStruct((BN, 128), jnp.int32)]
    + [jax.ShapeDtypeStruct((BN, DIM), F32)] * 3,
)

NC, NS = 2, 16
NW = NC * NS
RPW = E // NW
CH = RPW // 128


@functools.lru_cache(maxsize=None)
def _make_gather(with_pos):
    _sc_mesh = plsc.VectorSubcoreMesh(core_axis_name="c", subcore_axis_name="s",
                                      num_cores=NC, num_subcores=NS)

    def body(*refs):
        if with_pos:
            ptab, idx_hbm, pg, idx_v, pbuf, sem = refs
            tabs, bufs, outs = [ptab], [pbuf], [pg]
        else:
            ktab, vtab, idx_hbm, kg, vg, idx_v, kbuf, vbuf, sem = refs
            tabs, bufs, outs = [ktab, vtab], [kbuf, vbuf], [kg, vg]
        wid = lax.axis_index("s") * NC + lax.axis_index("c")
        row0 = wid * RPW
        pltpu.sync_copy(idx_hbm.at[wid], idx_v)
        copies = []
        for j in range(CH):
            ij = idx_v.at[j]
            dst = pl.ds(j * 128, 128)
            for tab, buf in zip(tabs, bufs):
                copies.append(pltpu.async_copy(tab.at[ij], buf.at[dst], sem))
        for cp in copies:
            cp.wait()
        for buf, out in zip(bufs, outs):
            pltpu.sync_copy(buf, out.at[pl.ds(row0, RPW)])

    n = 1 if with_pos else 2
    out_type = [jax.ShapeDtypeStruct((E, DIM), F32)] * n
    scratch = ([pltpu.VMEM((8, 128), jnp.int32)]
               + [pltpu.VMEM((RPW, DIM), F32)] * n
               + [pltpu.SemaphoreType.DMA])
    return pl.kernel(body, out_type=out_type, mesh=_sc_mesh,
                     scratch_types=scratch)


def _gather_pos(p, idx2d):
    return _make_gather(True)(p, idx2d)[0]


def _gather_kv(k, v, idx2d):
    return _make_gather(False)(k, v, idx2d)

QL = 512
NL = BN // QL
NPB = N // QL


def _layer_body(emit_qkv, *refs):
    if emit_qkv:
        (q_ref, kg_ref, vg_ref, pg_ref, posr_ref,
         wp1_ref, bp1_ref, wp2_ref, bp2_ref,
         wa1_ref, ba1_ref, wa2_ref, ba2_ref,
         wq_ref, wk_ref, wv_ref,
         qo_ref, ko_ref, vo_ref) = refs
    else:
        (q_ref, kg_ref, vg_ref, pg_ref, posr_ref,
         wp1_ref, bp1_ref, wp2_ref, bp2_ref,
         wa1_ref, ba1_ref, wa2_ref, ba2_ref,
         wfc_ref, bfc_ref,
         pool_ref, head_ref) = refs
    q = q_ref[...]
    posr = posr_ref[...]
    wp1 = wp1_ref[...]
    bp1 = bp1_ref[...]
    wp2 = wp2_ref[...]
    bp2 = bp2_ref[...]
    wa1 = wa1_ref[...]
    ba1 = ba1_ref[...]
    wa2 = wa2_ref[...]
    ba2 = ba2_ref[...]
    sims = []
    ves = []
    for t in range(K):
        rel = posr - pg_ref[t][:, 0:3]
        p1 = jnp.maximum(
            jnp.dot(rel, wp1, preferred_element_type=F32) + bp1, 0.0)
        pe = jnp.dot(p1, wp2, preferred_element_type=F32) + bp2
        u = q - kg_ref[t] + pe
        s = jnp.maximum(
            jnp.dot(u, wa1, preferred_element_type=F32) + ba1, 0.0)
        sims.append(jnp.dot(s, wa2, preferred_element_type=F32) + ba2)
        ves.append(vg_ref[t] + pe)
    m = jnp.maximum(jnp.maximum(sims[0], sims[1]), sims[2])
    es = [jnp.exp(sv - m) for sv in sims]
    den = es[0] + es[1] + es[2]
    h = (es[0] * ves[0] + es[1] * ves[1] + es[2] * ves[2]) / den
    if emit_qkv:
        qo_ref[...] = jnp.dot(h, wq_ref[...], preferred_element_type=F32)
        ko_ref[...] = jnp.dot(h, wk_ref[...], preferred_element_type=F32)
        vo_ref[...] = jnp.dot(h, wv_ref[...], preferred_element_type=F32)
    else:
        i = pl.program_id(0)
        part = jnp.sum(h, axis=0, keepdims=True)
        biota = lax.broadcasted_iota(jnp.int32, (B, 1, DIM), 0)
        upd = jnp.where(biota == i // NPB, part[None], 0.0)

        @pl.when(i == 0)
        def _():
            pool_ref[...] = jnp.zeros_like(pool_ref)

        pool_ref[...] += upd

        @pl.when(i == NL - 1)
        def _():
            pooled = pool_ref[...].reshape(B, DIM) * (1.0 / N)
            head_ref[...] = (jnp.dot(pooled, wfc_ref[...],
                                     preferred_element_type=F32)
                             + bfc_ref[...]).reshape(B, 1, 128)


def _make_layer(emit_qkv):
    in_specs = [
        pl.BlockSpec((QL, DIM), lambda i: (i, 0)),
        pl.BlockSpec((K, QL, DIM), lambda i: (0, i, 0)),
        pl.BlockSpec((K, QL, DIM), lambda i: (0, i, 0)),
        pl.BlockSpec((K, QL, PPAD), lambda i: (0, i, 0)),
        pl.BlockSpec((QL, 3), lambda i: (i, 0)),
        pl.BlockSpec((3, POS_H), lambda i: (0, 0)),
        pl.BlockSpec((1, POS_H), lambda i: (0, 0)),
        pl.BlockSpec((POS_H, DIM), lambda i: (0, 0)),
        pl.BlockSpec((1, DIM), lambda i: (0, 0)),
        pl.BlockSpec((DIM, ATT_H), lambda i: (0, 0)),
        pl.BlockSpec((1, ATT_H), lambda i: (0, 0)),
        pl.BlockSpec((ATT_H, DIM), lambda i: (0, 0)),
        pl.BlockSpec((1, DIM), lambda i: (0, 0)),
    ]
    if emit_qkv:
        in_specs += [pl.BlockSpec((DIM, DIM), lambda i: (0, 0))] * 3
        out_specs = [pl.BlockSpec((QL, DIM), lambda i: (i, 0))] * 3
        out_shape = [jax.ShapeDtypeStruct((BN, DIM), F32)] * 3
    else:
        in_specs += [
            pl.BlockSpec((DIM, 128), lambda i: (0, 0)),
            pl.BlockSpec((1, 128), lambda i: (0, 0)),
        ]
        out_specs = [
            pl.BlockSpec((B, 1, DIM), lambda i: (0, 0, 0)),
            pl.BlockSpec((B, 1, 128), lambda i: (0, 0, 0)),
        ]
        out_shape = [jax.ShapeDtypeStruct((B, 1, DIM), F32),
                     jax.ShapeDtypeStruct((B, 1, 128), F32)]
    return pl.pallas_call(
        functools.partial(_layer_body, emit_qkv),
        grid=(NL,),
        in_specs=in_specs,
        out_specs=out_specs,
        out_shape=out_shape,
    )


_layer_mid = _make_layer(True)
_layer_last = _make_layer(False)


def kernel(x, pos, W_enc, b_enc, Wqkv, Wp1, bp1, Wp2, bp2, Wa1, ba1,
           Wa2, ba2, W_fc, b_fc):
    pos3 = pos.reshape(BN, 3)
    posc = jnp.transpose(pos, (0, 2, 1))
    Wq = Wqkv[:, :, :DIM]
    Wk = Wqkv[:, :, DIM:2 * DIM]
    Wv = Wqkv[:, :, 2 * DIM:]

    idx_cols, q, k, v = _knn_call(pos3, posc, x.reshape(BN, IN_F), W_enc,
                                  b_enc.reshape(1, DIM), Wq[0], Wk[0], Wv[0])
    idx2d = jnp.pad(idx_cols[:, :K].T.reshape(NW, CH, 128),
                    ((0, 0), (0, 8 - CH), (0, 0)))

    ptab = jnp.pad(pos3, ((0, 0), (0, PPAD - 3)))
    pg = _gather_pos(ptab, idx2d)
    pg3 = pg.reshape(K, BN, PPAD)

    wfc_pad = jnp.pad(W_fc, ((0, 0), (0, 128 - 1)))
    bfc_pad = jnp.pad(b_fc.reshape(1, 1), ((0, 0), (0, 128 - 1)))

    for l in range(L):
        kg, vg = _gather_kv(k, v, idx2d)
        args = (q, kg.reshape(K, BN, DIM), vg.reshape(K, BN, DIM),
                pg3, pos3,
                Wp1[l], bp1[l].reshape(1, POS_H), Wp2[l],
                bp2[l].reshape(1, DIM), Wa1[l], ba1[l].reshape(1, ATT_H),
                Wa2[l], ba2[l].reshape(1, DIM))
        if l < L - 1:
            q, k, v = _layer_mid(*(args + (Wq[l + 1], Wk[l + 1], Wv[l + 1])))
        else:
            _, head = _layer_last(*(args + (wfc_pad, bfc_pad)))
    return head[:, 0, 0:1]


# speedup vs baseline: 20.5758x; 1.1067x over previous
import functools

import jax
import jax.numpy as jnp
from jax import lax
from jax.experimental import pallas as pl
from jax.experimental.pallas import tpu as pltpu
from jax.experimental.pallas import tpu_sc as plsc

B, N, K = 2, 2048, 3
DIM, IN_F = 128, 128
POS_H, ATT_H = 64, 512
L = 3
BN = B * N
E = K * BN
PPAD = 128

F32 = jnp.float32
U32 = jnp.uint32


def _pack_kv(kk, vv):
    kb = lax.bitcast_convert_type(kk, U32)
    vb = lax.bitcast_convert_type(vv, U32)
    kr = (kb + U32(0x7FFF) + ((kb >> 16) & U32(1))) >> 16
    vr = (vb + U32(0x7FFF) + ((vb >> 16) & U32(1))) >> 16
    return (kr << 16) | vr


def _unpack_k(w):
    return lax.bitcast_convert_type(w & U32(0xFFFF0000), F32)


def _unpack_v(w):
    return lax.bitcast_convert_type(w << 16, F32)

QB = 256
NQ = N // QB


def _knn_body(posq_ref, posc_ref, x_ref, we_ref, be_ref, wq_ref, wk_ref,
              wv_ref, idx_ref, q_ref, kv_ref):
    b = pl.program_id(0)
    q = posq_ref[...]
    pc = posc_ref[0]
    d = None
    for c in range(3):
        rel = q[:, c:c + 1] - pc[c:c + 1, :]
        sq = rel * rel
        d = sq if d is None else d + sq
    iota = lax.broadcasted_iota(jnp.int32, (QB, N), 1)
    lane = lax.broadcasted_iota(jnp.int32, (QB, 128), 1)
    base = b * N
    acc = jnp.zeros((QB, 128), jnp.int32)
    for t in range(K):
        it = jnp.argmin(d, axis=1).reshape(QB, 1)
        d = jnp.where(iota == it, jnp.float32(jnp.inf), d)
        acc = acc + jnp.where(lane == t, it + base, 0)
    idx_ref[...] = acc
    h = (jnp.dot(x_ref[...], we_ref[...], preferred_element_type=F32)
         + be_ref[...])
    q_ref[...] = jnp.dot(h, wq_ref[...], preferred_element_type=F32)
    kk = jnp.dot(h, wk_ref[...], preferred_element_type=F32)
    vv = jnp.dot(h, wv_ref[...], preferred_element_type=F32)
    kv_ref[...] = _pack_kv(kk, vv)


_knn_call = pl.pallas_call(
    _knn_body,
    grid=(B, NQ),
    in_specs=[
        pl.BlockSpec((QB, 3), lambda b, i: (b * NQ + i, 0)),
        pl.BlockSpec((1, 3, N), lambda b, i: (b, 0, 0)),
        pl.BlockSpec((QB, IN_F), lambda b, i: (b * NQ + i, 0)),
        pl.BlockSpec((IN_F, DIM), lambda b, i: (0, 0)),
        pl.BlockSpec((1, DIM), lambda b, i: (0, 0)),
        pl.BlockSpec((DIM, DIM), lambda b, i: (0, 0)),
        pl.BlockSpec((DIM, DIM), lambda b, i: (0, 0)),
        pl.BlockSpec((DIM, DIM), lambda b, i: (0, 0)),
    ],
    out_specs=[
        pl.BlockSpec((QB, 128), lambda b, i: (b * NQ + i, 0)),
        pl.BlockSpec((QB, DIM), lambda b, i: (b * NQ + i, 0)),
        pl.BlockSpec((QB, DIM), lambda b, i: (b * NQ + i, 0)),
    ],
    out_shape=[jax.ShapeDtypeStruct((BN, 128), jnp.int32),
               jax.ShapeDtypeStruct((BN, DIM), F32),
               jax.ShapeDtypeStruct((BN, DIM), U32)],
)

NC, NS = 2, 16
NW = NC * NS
RPW = E // NW
CH = RPW // 128


@functools.lru_cache(maxsize=None)
def _make_gather(with_pos):
    _sc_mesh = plsc.VectorSubcoreMesh(core_axis_name="c", subcore_axis_name="s",
                                      num_cores=NC, num_subcores=NS)

    def body(*refs):
        if with_pos:
            ptab, idx_hbm, pg, idx_v, pbuf, sem = refs
            tabs, bufs, outs = [ptab], [pbuf], [pg]
        else:
            kvtab, idx_hbm, kvg, idx_v, kvbuf, sem = refs
            tabs, bufs, outs = [kvtab], [kvbuf], [kvg]
        wid = lax.axis_index("s") * NC + lax.axis_index("c")
        row0 = wid * RPW
        pltpu.sync_copy(idx_hbm.at[wid], idx_v)
        copies = []
        for j in range(CH):
            ij = idx_v.at[j]
            dst = pl.ds(j * 128, 128)
            for tab, buf in zip(tabs, bufs):
                copies.append(pltpu.async_copy(tab.at[ij], buf.at[dst], sem))
        for cp in copies:
            cp.wait()
        for buf, out in zip(bufs, outs):
            pltpu.sync_copy(buf, out.at[pl.ds(row0, RPW)])

    dt = F32 if with_pos else U32
    out_type = [jax.ShapeDtypeStruct((E, DIM), dt)]
    scratch = ([pltpu.VMEM((8, 128), jnp.int32),
                pltpu.VMEM((RPW, DIM), dt),
                pltpu.SemaphoreType.DMA])
    return pl.kernel(body, out_type=out_type, mesh=_sc_mesh,
                     scratch_types=scratch)


def _gather_pos(p, idx2d):
    return _make_gather(True)(p, idx2d)[0]


def _gather_kv(kv, idx2d):
    return _make_gather(False)(kv, idx2d)[0]

QL = 512
NL = BN // QL
NPB = N // QL


def _layer_body(emit_qkv, *refs):
    if emit_qkv:
        (q_ref, kvg_ref, pg_ref, posr_ref,
         wp1_ref, bp1_ref, wp2_ref, bp2_ref,
         wa1_ref, ba1_ref, wa2_ref, ba2_ref,
         wq_ref, wk_ref, wv_ref,
         qo_ref, kvo_ref) = refs
    else:
        (q_ref, kvg_ref, pg_ref, posr_ref,
         wp1_ref, bp1_ref, wp2_ref, bp2_ref,
         wa1_ref, ba1_ref, wa2_ref, ba2_ref,
         wfc_ref, bfc_ref,
         pool_ref, head_ref) = refs
    q = q_ref[...]
    posr = posr_ref[...]
    wp1 = wp1_ref[...]
    bp1 = bp1_ref[...]
    wp2 = wp2_ref[...]
    bp2 = bp2_ref[...]
    wa1 = wa1_ref[...]
    ba1 = ba1_ref[...]
    wa2 = wa2_ref[...]
    ba2 = ba2_ref[...]
    sims = []
    ves = []
    for t in range(K):
        rel = posr - pg_ref[t][:, 0:3]
        p1 = jnp.maximum(
            jnp.dot(rel, wp1, preferred_element_type=F32) + bp1, 0.0)
        pe = jnp.dot(p1, wp2, preferred_element_type=F32) + bp2
        w = kvg_ref[t]
        u = q - _unpack_k(w) + pe
        s = jnp.maximum(
            jnp.dot(u, wa1, preferred_element_type=F32) + ba1, 0.0)
        sims.append(jnp.dot(s, wa2, preferred_element_type=F32) + ba2)
        ves.append(_unpack_v(w) + pe)
    m = jnp.maximum(jnp.maximum(sims[0], sims[1]), sims[2])
    es = [jnp.exp(sv - m) for sv in sims]
    den = es[0] + es[1] + es[2]
    h = (es[0] * ves[0] + es[1] * ves[1] + es[2] * ves[2]) / den
    if emit_qkv:
        qo_ref[...] = jnp.dot(h, wq_ref[...], preferred_element_type=F32)
        kk = jnp.dot(h, wk_ref[...], preferred_element_type=F32)
        vv = jnp.dot(h, wv_ref[...], preferred_element_type=F32)
        kvo_ref[...] = _pack_kv(kk, vv)
    else:
        i = pl.program_id(0)
        part = jnp.sum(h, axis=0, keepdims=True)
        biota = lax.broadcasted_iota(jnp.int32, (B, 1, DIM), 0)
        upd = jnp.where(biota == i // NPB, part[None], 0.0)

        @pl.when(i == 0)
        def _():
            pool_ref[...] = jnp.zeros_like(pool_ref)

        pool_ref[...] += upd

        @pl.when(i == NL - 1)
        def _():
            pooled = pool_ref[...].reshape(B, DIM) * (1.0 / N)
            head_ref[...] = (jnp.dot(pooled, wfc_ref[...],
                                     preferred_element_type=F32)
                             + bfc_ref[...]).reshape(B, 1, 128)


def _make_layer(emit_qkv):
    in_specs = [
        pl.BlockSpec((QL, DIM), lambda i: (i, 0)),
        pl.BlockSpec((K, QL, DIM), lambda i: (0, i, 0)),
        pl.BlockSpec((K, QL, PPAD), lambda i: (0, i, 0)),
        pl.BlockSpec((QL, 3), lambda i: (i, 0)),
        pl.BlockSpec((3, POS_H), lambda i: (0, 0)),
        pl.BlockSpec((1, POS_H), lambda i: (0, 0)),
        pl.BlockSpec((POS_H, DIM), lambda i: (0, 0)),
        pl.BlockSpec((1, DIM), lambda i: (0, 0)),
        pl.BlockSpec((DIM, ATT_H), lambda i: (0, 0)),
        pl.BlockSpec((1, ATT_H), lambda i: (0, 0)),
        pl.BlockSpec((ATT_H, DIM), lambda i: (0, 0)),
        pl.BlockSpec((1, DIM), lambda i: (0, 0)),
    ]
    if emit_qkv:
        in_specs += [pl.BlockSpec((DIM, DIM), lambda i: (0, 0))] * 3
        out_specs = [pl.BlockSpec((QL, DIM), lambda i: (i, 0)),
                     pl.BlockSpec((QL, DIM), lambda i: (i, 0))]
        out_shape = [jax.ShapeDtypeStruct((BN, DIM), F32),
                     jax.ShapeDtypeStruct((BN, DIM), U32)]
    else:
        in_specs += [
            pl.BlockSpec((DIM, 128), lambda i: (0, 0)),
            pl.BlockSpec((1, 128), lambda i: (0, 0)),
        ]
        out_specs = [
            pl.BlockSpec((B, 1, DIM), lambda i: (0, 0, 0)),
            pl.BlockSpec((B, 1, 128), lambda i: (0, 0, 0)),
        ]
        out_shape = [jax.ShapeDtypeStruct((B, 1, DIM), F32),
                     jax.ShapeDtypeStruct((B, 1, 128), F32)]
    return pl.pallas_call(
        functools.partial(_layer_body, emit_qkv),
        grid=(NL,),
        in_specs=in_specs,
        out_specs=out_specs,
        out_shape=out_shape,
    )


_layer_mid = _make_layer(True)
_layer_last = _make_layer(False)


def kernel(x, pos, W_enc, b_enc, Wqkv, Wp1, bp1, Wp2, bp2, Wa1, ba1,
           Wa2, ba2, W_fc, b_fc):
    pos3 = pos.reshape(BN, 3)
    posc = jnp.transpose(pos, (0, 2, 1))
    Wq = Wqkv[:, :, :DIM]
    Wk = Wqkv[:, :, DIM:2 * DIM]
    Wv = Wqkv[:, :, 2 * DIM:]

    idx_cols, q, kv = _knn_call(pos3, posc, x.reshape(BN, IN_F), W_enc,
                                b_enc.reshape(1, DIM), Wq[0], Wk[0], Wv[0])
    idx2d = jnp.pad(idx_cols[:, :K].T.reshape(NW, CH, 128),
                    ((0, 0), (0, 8 - CH), (0, 0)))

    ptab = jnp.pad(pos3, ((0, 0), (0, PPAD - 3)))
    pg = _gather_pos(ptab, idx2d)
    pg3 = pg.reshape(K, BN, PPAD)

    wfc_pad = jnp.pad(W_fc, ((0, 0), (0, 128 - 1)))
    bfc_pad = jnp.pad(b_fc.reshape(1, 1), ((0, 0), (0, 128 - 1)))

    for l in range(L):
        kvg = _gather_kv(kv, idx2d)
        args = (q, kvg.reshape(K, BN, DIM),
                pg3, pos3,
                Wp1[l], bp1[l].reshape(1, POS_H), Wp2[l],
                bp2[l].reshape(1, DIM), Wa1[l], ba1[l].reshape(1, ATT_H),
                Wa2[l], ba2[l].reshape(1, DIM))
        if l < L - 1:
            q, kv = _layer_mid(*(args + (Wq[l + 1], Wk[l + 1], Wv[l + 1])))
        else:
            _, head = _layer_last(*(args + (wfc_pad, bfc_pad)))
    return head[:, 0, 0:1]
